# baseline (speedup 1.0000x reference)
import functools

import jax
import jax.numpy as jnp
from jax import lax
from jax.experimental import pallas as pl
from jax.experimental.pallas import tpu as pltpu
from jax.experimental.pallas import tpu_sc as plsc

N_NODES = 10000
N_EDGES = 320000
D = 128

NC, NS = 2, 16
NW = NC * NS
CHUNK = 128
E_PAD = 327680
TOT_CHUNK = E_PAD // CHUNK
K0 = 160
K1 = 0
SG = 32
ROWS_PER_TILE = 632
ACC_ROWS = NS * ROWS_PER_TILE
PAD_ROW = N_NODES

_sc_mesh = plsc.VectorSubcoreMesh(core_axis_name="c", subcore_axis_name="s")


@functools.partial(
    pl.kernel,
    out_type=jax.ShapeDtypeStruct((NC, ACC_ROWS, D), jnp.float32),
    mesh=_sc_mesh,
    scratch_types=[
        pltpu.VMEM((SG, CHUNK), jnp.int32),
        pltpu.VMEM((SG, CHUNK), jnp.int32),
        pltpu.VMEM((CHUNK, D), jnp.float32),
        pltpu.VMEM((CHUNK, D), jnp.float32),
        pltpu.VMEM_SHARED((ACC_ROWS, D), jnp.float32),
        pltpu.SemaphoreType.DMA,
        pltpu.SemaphoreType.DMA,
    ],
)
def _sc_aggregate(x_hbm, src_hbm, dst_hbm, zeros_hbm, out_hbm,
                  src_v, dst_v, rows0, rows1, acc, sem0, sem1):
    c = lax.axis_index("c")
    s = lax.axis_index("s")
    row0 = s * ROWS_PER_TILE
    pltpu.sync_copy(zeros_hbm.at[pl.ds(0, ROWS_PER_TILE)],
                    acc.at[pl.ds(row0, ROWS_PER_TILE)])
    plsc.subcore_barrier()

    first = jnp.where(c == 0, s * K0, NS * K0 + s * K1)
    nstage = jnp.where(c == 0, K0 // SG, K1 // SG)

    def stage(t, carry):
        base = first + t * SG
        pltpu.sync_copy(src_hbm.at[pl.ds(base, SG)], src_v)
        pltpu.sync_copy(dst_hbm.at[pl.ds(base, SG)], dst_v)
        pltpu.async_copy(x_hbm.at[src_v.at[0]], rows0, sem0)
        pltpu.async_copy(x_hbm.at[src_v.at[1]], rows1, sem1)

        def body(k, carry2):
            i = 2 * k
            for b, rows, sem in ((0, rows0, sem0), (1, rows1, sem1)):
                j = i + b
                pltpu.make_async_copy(
                    x_hbm.at[src_v.at[j]], rows, sem).wait()
                pltpu.sync_copy(rows, acc.at[dst_v.at[j]], add=True)

                @pl.when(j + 2 < SG)
                def _():
                    pltpu.async_copy(x_hbm.at[src_v.at[j + 2]], rows, sem)
            return carry2

        lax.fori_loop(0, SG // 2, body, 0)
        return carry

    lax.fori_loop(0, nstage, stage, 0)
    plsc.subcore_barrier()
    pltpu.sync_copy(acc.at[pl.ds(row0, ROWS_PER_TILE)],
                    out_hbm.at[c].at[pl.ds(row0, ROWS_PER_TILE)])


def _tc_mlp(p_ref, w1_ref, b1_ref, w2_ref, t_ref):
    x = p_ref[0] + p_ref[1]
    h = lax.dot_general(x, w1_ref[...], (((1,), (1,)), ((), ())),
                        preferred_element_type=jnp.float32)
    h = jnp.maximum(h + b1_ref[...], 0.0)
    t_ref[...] = lax.dot_general(h, w2_ref[...], (((1,), (1,)), ((), ())),
                                 preferred_element_type=jnp.float32)


def _tc_bias_relu(q_ref, b2_ref, o_ref):
    o_ref[...] = jnp.maximum(q_ref[0] + q_ref[1] + b2_ref[...], 0.0)


_ROW_BLK = 1000
_N_BLK = N_NODES // _ROW_BLK


def kernel(feature, edge_index, W1, b1, W2, b2):
    src = edge_index[0].astype(jnp.int32)
    dst = edge_index[1].astype(jnp.int32)
    src = jnp.concatenate(
        [src, jnp.zeros((E_PAD - N_EDGES,), jnp.int32)]).reshape(
            TOT_CHUNK, CHUNK)
    pad_dst = PAD_ROW + (jnp.arange(E_PAD - N_EDGES, dtype=jnp.int32)
                         % (ACC_ROWS - N_NODES))
    dst = jnp.concatenate([dst, pad_dst]).reshape(TOT_CHUNK, CHUNK)
    zeros = jnp.zeros((ROWS_PER_TILE, D), jnp.float32)
    b1r = b1.reshape(1, -1)
    b2r = b2.reshape(1, -1)

    p = _sc_aggregate(feature, src, dst, zeros)

    t = pl.pallas_call(
        _tc_mlp,
        grid=(_N_BLK,),
        in_specs=[
            pl.BlockSpec((NC, _ROW_BLK, D), lambda i: (0, i, 0)),
            pl.BlockSpec(W1.shape, lambda i: (0, 0)),
            pl.BlockSpec(b1r.shape, lambda i: (0, 0)),
            pl.BlockSpec(W2.shape, lambda i: (0, 0)),
        ],
        out_specs=pl.BlockSpec((_ROW_BLK, D), lambda i: (i, 0)),
        out_shape=jax.ShapeDtypeStruct((N_NODES, D), jnp.float32),
    )(p, W1, b1r, W2)

    q = _sc_aggregate(t, src, dst, zeros)

    out = pl.pallas_call(
        _tc_bias_relu,
        grid=(_N_BLK,),
        in_specs=[
            pl.BlockSpec((NC, _ROW_BLK, D), lambda i: (0, i, 0)),
            pl.BlockSpec(b2r.shape, lambda i: (0, 0)),
        ],
        out_specs=pl.BlockSpec((_ROW_BLK, D), lambda i: (i, 0)),
        out_shape=jax.ShapeDtypeStruct((N_NODES, D), jnp.float32),
    )(q, b2r)
    return out


# speedup vs baseline: 1.3492x; 1.3492x over previous
import functools

import jax
import jax.numpy as jnp
from jax import lax
from jax.experimental import pallas as pl
from jax.experimental.pallas import tpu as pltpu
from jax.experimental.pallas import tpu_sc as plsc

N_NODES = 10000
N_EDGES = 320000
D = 128

NC, NS = 2, 16
NW = NC * NS
CHUNK = 128
E_PAD = 327680
TOT_CHUNK = E_PAD // CHUNK
K0 = 80
K1 = 80
SG = 40
EPW_REAL = N_EDGES // NW
EPW_PAD = E_PAD // NW - EPW_REAL
ROWS_PER_TILE = 632
ACC_ROWS = NS * ROWS_PER_TILE
PAD_ROW = N_NODES

_sc_mesh = plsc.VectorSubcoreMesh(core_axis_name="c", subcore_axis_name="s")


@functools.partial(
    pl.kernel,
    out_type=jax.ShapeDtypeStruct((NC, ACC_ROWS, D), jnp.float32),
    mesh=_sc_mesh,
    scratch_types=[
        pltpu.VMEM((SG, CHUNK), jnp.int32),
        pltpu.VMEM((SG, CHUNK), jnp.int32),
        pltpu.VMEM((CHUNK, D), jnp.float32),
        pltpu.VMEM((CHUNK, D), jnp.float32),
        pltpu.VMEM_SHARED((ACC_ROWS, D), jnp.float32),
        pltpu.SemaphoreType.DMA,
        pltpu.SemaphoreType.DMA,
    ],
)
def _sc_aggregate(x_hbm, src_hbm, dst_hbm, zeros_hbm, out_hbm,
                  src_v, dst_v, rows0, rows1, acc, sem0, sem1):
    c = lax.axis_index("c")
    s = lax.axis_index("s")
    row0 = s * ROWS_PER_TILE
    pltpu.sync_copy(zeros_hbm.at[pl.ds(0, ROWS_PER_TILE)],
                    acc.at[pl.ds(row0, ROWS_PER_TILE)])
    plsc.subcore_barrier()

    first = jnp.where(c == 0, s * K0, NS * K0 + s * K1)
    nstage = jnp.where(c == 0, K0 // SG, K1 // SG)

    def stage(t, carry):
        base = first + t * SG
        pltpu.sync_copy(src_hbm.at[pl.ds(base, SG)], src_v)
        pltpu.sync_copy(dst_hbm.at[pl.ds(base, SG)], dst_v)
        pltpu.async_copy(x_hbm.at[src_v.at[0]], rows0, sem0)
        pltpu.async_copy(x_hbm.at[src_v.at[1]], rows1, sem1)

        def body(k, carry2):
            i = 2 * k
            for b, rows, sem in ((0, rows0, sem0), (1, rows1, sem1)):
                j = i + b
                pltpu.make_async_copy(
                    x_hbm.at[src_v.at[j]], rows, sem).wait()
                pltpu.sync_copy(rows, acc.at[dst_v.at[j]], add=True)

                @pl.when(j + 2 < SG)
                def _():
                    pltpu.async_copy(x_hbm.at[src_v.at[j + 2]], rows, sem)
            return carry2

        lax.fori_loop(0, SG // 2, body, 0)
        return carry

    lax.fori_loop(0, nstage, stage, 0)
    plsc.subcore_barrier()
    pltpu.sync_copy(acc.at[pl.ds(row0, ROWS_PER_TILE)],
                    out_hbm.at[c].at[pl.ds(row0, ROWS_PER_TILE)])


def _tc_mlp(p_ref, w1_ref, b1_ref, w2_ref, t_ref):
    x = p_ref[0] + p_ref[1]
    h = lax.dot_general(x, w1_ref[...], (((1,), (1,)), ((), ())),
                        preferred_element_type=jnp.float32)
    h = jnp.maximum(h + b1_ref[...], 0.0)
    t_ref[...] = lax.dot_general(h, w2_ref[...], (((1,), (1,)), ((), ())),
                                 preferred_element_type=jnp.float32)


def _tc_bias_relu(q_ref, b2_ref, o_ref):
    o_ref[...] = jnp.maximum(q_ref[0] + q_ref[1] + b2_ref[...], 0.0)


_ROW_BLK = 1000
_N_BLK = N_NODES // _ROW_BLK


def kernel(feature, edge_index, W1, b1, W2, b2):
    src = edge_index[0].astype(jnp.int32)
    dst = edge_index[1].astype(jnp.int32)
    pad_src = jnp.zeros((NW, EPW_PAD), jnp.int32)
    pad_dst = PAD_ROW + (jnp.arange(EPW_PAD, dtype=jnp.int32)
                         % (ACC_ROWS - N_NODES))
    pad_dst = jnp.broadcast_to(pad_dst, (NW, EPW_PAD))
    src = jnp.concatenate(
        [src.reshape(NW, EPW_REAL), pad_src], axis=1).reshape(
            TOT_CHUNK, CHUNK)
    dst = jnp.concatenate(
        [dst.reshape(NW, EPW_REAL), pad_dst], axis=1).reshape(
            TOT_CHUNK, CHUNK)
    zeros = jnp.zeros((ROWS_PER_TILE, D), jnp.float32)
    b1r = b1.reshape(1, -1)
    b2r = b2.reshape(1, -1)

    p = _sc_aggregate(feature, src, dst, zeros)

    t = pl.pallas_call(
        _tc_mlp,
        grid=(_N_BLK,),
        in_specs=[
            pl.BlockSpec((NC, _ROW_BLK, D), lambda i: (0, i, 0)),
            pl.BlockSpec(W1.shape, lambda i: (0, 0)),
            pl.BlockSpec(b1r.shape, lambda i: (0, 0)),
            pl.BlockSpec(W2.shape, lambda i: (0, 0)),
        ],
        out_specs=pl.BlockSpec((_ROW_BLK, D), lambda i: (i, 0)),
        out_shape=jax.ShapeDtypeStruct((N_NODES, D), jnp.float32),
    )(p, W1, b1r, W2)

    q = _sc_aggregate(t, src, dst, zeros)

    out = pl.pallas_call(
        _tc_bias_relu,
        grid=(_N_BLK,),
        in_specs=[
            pl.BlockSpec((NC, _ROW_BLK, D), lambda i: (0, i, 0)),
            pl.BlockSpec(b2r.shape, lambda i: (0, 0)),
        ],
        out_specs=pl.BlockSpec((_ROW_BLK, D), lambda i: (i, 0)),
        out_shape=jax.ShapeDtypeStruct((N_NODES, D), jnp.float32),
    )(q, b2r)
    return out


# speedup vs baseline: 4.1490x; 3.0752x over previous
import functools

import jax
import jax.numpy as jnp
from jax import lax
from jax.experimental import pallas as pl
from jax.experimental.pallas import tpu as pltpu
from jax.experimental.pallas import tpu_sc as plsc

N_NODES = 10000
N_EDGES = 320000
D = 128

NC, NS = 2, 16
NW = NC * NS
CHUNK = 128
TOT_CHUNK = N_EDGES // CHUNK
SG = 40
SGW = SG + 8
IDX_ROWS = 2560
ROWS_PER_TILE = 632
ACC_ROWS = NS * ROWS_PER_TILE

_sc_mesh = plsc.VectorSubcoreMesh(core_axis_name="c", subcore_axis_name="s")


@functools.partial(
    pl.kernel,
    out_type=jax.ShapeDtypeStruct((NC, ACC_ROWS, D), jnp.float32),
    mesh=_sc_mesh,
    scratch_types=[
        pltpu.VMEM((SGW, CHUNK), jnp.int32),
        pltpu.VMEM((SGW, CHUNK), jnp.int32),
        pltpu.VMEM((CHUNK, D), jnp.float32),
        pltpu.VMEM((CHUNK, D), jnp.float32),
        pltpu.VMEM_SHARED((ACC_ROWS, D), jnp.float32),
        pltpu.SemaphoreType.DMA,
        pltpu.SemaphoreType.DMA,
    ],
)
def _sc_aggregate(x_hbm, src_hbm, dst_hbm, zeros_hbm, out_hbm,
                  src_v, dst_v, rows0, rows1, acc, sem0, sem1):
    c = lax.axis_index("c")
    s = lax.axis_index("s")
    row0 = s * ROWS_PER_TILE
    pltpu.sync_copy(zeros_hbm.at[pl.ds(0, ROWS_PER_TILE)],
                    acc.at[pl.ds(row0, ROWS_PER_TILE)])
    plsc.subcore_barrier()

    wid = c * NS + s
    first = 78 * wid + (wid + 7) // 8
    nchunk = 78 + jnp.where(wid % 8 == 0, 1, 0)

    def stage(start, size):
        aligned = (start // 8) * 8
        off = start - aligned
        pltpu.sync_copy(src_hbm.at[pl.ds(aligned, SGW)], src_v)
        pltpu.sync_copy(dst_hbm.at[pl.ds(aligned, SGW)], dst_v)
        pltpu.async_copy(x_hbm.at[src_v.at[off]], rows0, sem0)
        pltpu.async_copy(x_hbm.at[src_v.at[off + 1]], rows1, sem1)

        def body(k, carry2):
            i = 2 * k
            for b, rows, sem in ((0, rows0, sem0), (1, rows1, sem1)):
                j = i + b

                @pl.when(j < size)
                def _():
                    pltpu.make_async_copy(
                        x_hbm.at[src_v.at[off + j]], rows, sem).wait()
                    pltpu.sync_copy(rows, acc.at[dst_v.at[off + j]],
                                    add=True)

                    @pl.when(j + 2 < size)
                    def _():
                        pltpu.async_copy(
                            x_hbm.at[src_v.at[off + j + 2]], rows, sem)
            return carry2

        lax.fori_loop(0, (size + 1) // 2, body, 0)

    stage(first, SG)
    stage(first + SG, nchunk - SG)
    plsc.subcore_barrier()
    pltpu.sync_copy(acc.at[pl.ds(row0, ROWS_PER_TILE)],
                    out_hbm.at[c].at[pl.ds(row0, ROWS_PER_TILE)])


def _tc_mlp(p_ref, w1_ref, b1_ref, w2_ref, t_ref):
    x = p_ref[0] + p_ref[1]
    h = lax.dot_general(x, w1_ref[...], (((1,), (1,)), ((), ())),
                        preferred_element_type=jnp.float32)
    h = jnp.maximum(h + b1_ref[...], 0.0)
    t_ref[...] = lax.dot_general(h, w2_ref[...], (((1,), (1,)), ((), ())),
                                 preferred_element_type=jnp.float32)


def _tc_bias_relu(q_ref, b2_ref, o_ref):
    o_ref[...] = jnp.maximum(q_ref[0] + q_ref[1] + b2_ref[...], 0.0)


_ROW_BLK = 1000
_N_BLK = N_NODES // _ROW_BLK


def kernel(feature, edge_index, W1, b1, W2, b2):
    src = edge_index[0].astype(jnp.int32)
    dst = edge_index[1].astype(jnp.int32)
    tail = jnp.zeros((IDX_ROWS - TOT_CHUNK, CHUNK), jnp.int32)
    src = jnp.concatenate([src.reshape(TOT_CHUNK, CHUNK), tail])
    dst = jnp.concatenate([dst.reshape(TOT_CHUNK, CHUNK), tail])
    zeros = jnp.zeros((ROWS_PER_TILE, D), jnp.float32)
    b1r = b1.reshape(1, -1)
    b2r = b2.reshape(1, -1)

    p = _sc_aggregate(feature, src, dst, zeros)

    t = pl.pallas_call(
        _tc_mlp,
        grid=(_N_BLK,),
        in_specs=[
            pl.BlockSpec((NC, _ROW_BLK, D), lambda i: (0, i, 0)),
            pl.BlockSpec(W1.shape, lambda i: (0, 0)),
            pl.BlockSpec(b1r.shape, lambda i: (0, 0)),
            pl.BlockSpec(W2.shape, lambda i: (0, 0)),
        ],
        out_specs=pl.BlockSpec((_ROW_BLK, D), lambda i: (i, 0)),
        out_shape=jax.ShapeDtypeStruct((N_NODES, D), jnp.float32),
    )(p, W1, b1r, W2)

    q = _sc_aggregate(t, src, dst, zeros)

    out = pl.pallas_call(
        _tc_bias_relu,
        grid=(_N_BLK,),
        in_specs=[
            pl.BlockSpec((NC, _ROW_BLK, D), lambda i: (0, i, 0)),
            pl.BlockSpec(b2r.shape, lambda i: (0, 0)),
        ],
        out_specs=pl.BlockSpec((_ROW_BLK, D), lambda i: (i, 0)),
        out_shape=jax.ShapeDtypeStruct((N_NODES, D), jnp.float32),
    )(q, b2r)
    return out


# speedup vs baseline: 4.3761x; 1.0547x over previous
import functools

import jax
import jax.numpy as jnp
from jax import lax
from jax.experimental import pallas as pl
from jax.experimental.pallas import tpu as pltpu
from jax.experimental.pallas import tpu_sc as plsc

N_NODES = 10000
N_EDGES = 320000
D = 128

NC, NS = 2, 16
NW = NC * NS
CHUNK = 128
TOT_CHUNK = N_EDGES // CHUNK
SG = 40
SGW = SG + 8
IDX_ROWS = 2560
ROWS_PER_TILE = 632
ACC_ROWS = NS * ROWS_PER_TILE

_sc_mesh = plsc.VectorSubcoreMesh(core_axis_name="c", subcore_axis_name="s")


@functools.partial(
    pl.kernel,
    out_type=jax.ShapeDtypeStruct((NC, ACC_ROWS, D), jnp.float32),
    mesh=_sc_mesh,
    scratch_types=[
        pltpu.VMEM((SGW, CHUNK), jnp.int32),
        pltpu.VMEM((SGW, CHUNK), jnp.int32),
        pltpu.VMEM((CHUNK, D), jnp.float32),
        pltpu.VMEM((CHUNK, D), jnp.float32),
        pltpu.VMEM_SHARED((ACC_ROWS, D), jnp.float32),
        pltpu.SemaphoreType.DMA,
        pltpu.SemaphoreType.DMA,
        pltpu.SemaphoreType.DMA,
    ],
)
def _sc_aggregate(x_hbm, src_hbm, dst_hbm, out_hbm,
                  src_v, dst_v, rows0, rows1, acc, sem0, sem1, ssem):
    c = lax.axis_index("c")
    s = lax.axis_index("s")
    row0 = s * ROWS_PER_TILE

    wid = c * NS + s
    first = 78 * wid + (wid + 7) // 8
    nchunk = 78 + jnp.where(wid % 8 == 0, 1, 0)
    aligned0 = (first // 8) * 8

    def zfill(r, carry):
        for cp in range(8):
            rows0[r, pl.ds(cp * 16, 16)] = jnp.zeros((16,), jnp.float32)
        return carry

    lax.fori_loop(0, CHUNK, zfill, 0)
    pltpu.async_copy(src_hbm.at[pl.ds(aligned0, SGW)], src_v, ssem)
    pltpu.async_copy(dst_hbm.at[pl.ds(aligned0, SGW)], dst_v, ssem)
    for k in range(ROWS_PER_TILE // CHUNK):
        pltpu.sync_copy(rows0, acc.at[pl.ds(row0 + k * CHUNK, CHUNK)])
    _rem = ROWS_PER_TILE % CHUNK
    pltpu.sync_copy(rows0.at[pl.ds(0, _rem)],
                    acc.at[pl.ds(row0 + ROWS_PER_TILE - _rem, _rem)])
    plsc.subcore_barrier()
    pltpu.make_async_copy(src_hbm.at[pl.ds(aligned0, SGW)], src_v,
                          ssem).wait()
    pltpu.make_async_copy(dst_hbm.at[pl.ds(aligned0, SGW)], dst_v,
                          ssem).wait()

    def stage(start, size, prestaged=False):
        aligned = (start // 8) * 8
        off = start - aligned
        if not prestaged:
            pltpu.sync_copy(src_hbm.at[pl.ds(aligned, SGW)], src_v)
            pltpu.sync_copy(dst_hbm.at[pl.ds(aligned, SGW)], dst_v)
        pltpu.async_copy(x_hbm.at[src_v.at[off]], rows0, sem0)
        pltpu.async_copy(x_hbm.at[src_v.at[off + 1]], rows1, sem1)

        def body(k, carry2):
            i = 2 * k
            for b, rows, sem in ((0, rows0, sem0), (1, rows1, sem1)):
                j = i + b

                @pl.when(j < size)
                def _():
                    pltpu.make_async_copy(
                        x_hbm.at[src_v.at[off + j]], rows, sem).wait()
                    pltpu.sync_copy(rows, acc.at[dst_v.at[off + j]],
                                    add=True)

                    @pl.when(j + 2 < size)
                    def _():
                        pltpu.async_copy(
                            x_hbm.at[src_v.at[off + j + 2]], rows, sem)
            return carry2

        lax.fori_loop(0, (size + 1) // 2, body, 0)

    stage(first, SG, prestaged=True)
    stage(first + SG, nchunk - SG)
    plsc.subcore_barrier()
    pltpu.sync_copy(acc.at[pl.ds(row0, ROWS_PER_TILE)],
                    out_hbm.at[c].at[pl.ds(row0, ROWS_PER_TILE)])


def _tc_mlp(p_ref, w1_ref, b1_ref, w2_ref, t_ref):
    x = p_ref[0] + p_ref[1]
    h = lax.dot_general(x, w1_ref[...], (((1,), (1,)), ((), ())),
                        preferred_element_type=jnp.float32)
    h = jnp.maximum(h + b1_ref[...], 0.0)
    t_ref[...] = lax.dot_general(h, w2_ref[...], (((1,), (1,)), ((), ())),
                                 preferred_element_type=jnp.float32)


def _tc_bias_relu(q_ref, b2_ref, o_ref):
    o_ref[...] = jnp.maximum(q_ref[0] + q_ref[1] + b2_ref[...], 0.0)


_ROW_BLK = 1000
_N_BLK = N_NODES // _ROW_BLK


def kernel(feature, edge_index, W1, b1, W2, b2):
    src = edge_index[0].astype(jnp.int32)
    dst = edge_index[1].astype(jnp.int32)
    tail = jnp.zeros((IDX_ROWS - TOT_CHUNK, CHUNK), jnp.int32)
    src = jnp.concatenate([src.reshape(TOT_CHUNK, CHUNK), tail])
    dst = jnp.concatenate([dst.reshape(TOT_CHUNK, CHUNK), tail])
    b1r = b1.reshape(1, -1)
    b2r = b2.reshape(1, -1)

    p = _sc_aggregate(feature, src, dst)

    t = pl.pallas_call(
        _tc_mlp,
        grid=(_N_BLK,),
        in_specs=[
            pl.BlockSpec((NC, _ROW_BLK, D), lambda i: (0, i, 0)),
            pl.BlockSpec(W1.shape, lambda i: (0, 0)),
            pl.BlockSpec(b1r.shape, lambda i: (0, 0)),
            pl.BlockSpec(W2.shape, lambda i: (0, 0)),
        ],
        out_specs=pl.BlockSpec((_ROW_BLK, D), lambda i: (i, 0)),
        out_shape=jax.ShapeDtypeStruct((N_NODES, D), jnp.float32),
    )(p, W1, b1r, W2)

    q = _sc_aggregate(t, src, dst)

    out = pl.pallas_call(
        _tc_bias_relu,
        grid=(_N_BLK,),
        in_specs=[
            pl.BlockSpec((NC, _ROW_BLK, D), lambda i: (0, i, 0)),
            pl.BlockSpec(b2r.shape, lambda i: (0, 0)),
        ],
        out_specs=pl.BlockSpec((_ROW_BLK, D), lambda i: (i, 0)),
        out_shape=jax.ShapeDtypeStruct((N_NODES, D), jnp.float32),
    )(q, b2r)
    return out
